# baseline (speedup 1.0000x reference)
import jax
import jax.numpy as jnp
from jax import lax
from jax.experimental import pallas as pl
from jax.experimental.pallas import tpu as pltpu

_TM = 576


@jax.jit
def kernel(ego_embeddings, A_in, W, b):
    n, in_dim = ego_embeddings.shape
    out_dim = W.shape[0]
    b2 = b.reshape(1, out_dim)
    nsteps = pl.cdiv(n, _TM)
    npad = nsteps * _TM

    def body(a_ref, e_ref, w_ref, b_ref, out_ref, epad_ref):
        i = pl.program_id(0)

        @pl.when(i == 0)
        def _():
            epad_ref[pl.ds(0, n), :] = e_ref[...]
            epad_ref[pl.ds(n, npad - n), :] = jnp.zeros(
                (npad - n, in_dim), jnp.float32)

        side = jnp.dot(a_ref[...], e_ref[...],
                       preferred_element_type=jnp.float32)
        h = side + epad_ref[pl.ds(i * _TM, _TM), :]
        o = lax.dot_general(h, w_ref[...], (((1,), (1,)), ((), ())),
                            preferred_element_type=jnp.float32)
        o = o + b_ref[...]
        out_ref[...] = jnp.where(o >= 0, o, 0.01 * o)

    return pl.pallas_call(
        body,
        grid=(nsteps,),
        in_specs=[
            pl.BlockSpec((_TM, n), lambda i: (i, 0)),
            pl.BlockSpec((n, in_dim), lambda i: (0, 0)),
            pl.BlockSpec((out_dim, in_dim), lambda i: (0, 0)),
            pl.BlockSpec((1, out_dim), lambda i: (0, 0)),
        ],
        out_specs=pl.BlockSpec((_TM, out_dim), lambda i: (i, 0)),
        out_shape=jax.ShapeDtypeStruct((n, out_dim), jnp.float32),
        scratch_shapes=[pltpu.VMEM((npad, in_dim), jnp.float32)],
        compiler_params=pltpu.CompilerParams(
            dimension_semantics=("arbitrary",),
        ),
    )(A_in, ego_embeddings, W, b2)
